# baseline (speedup 1.0000x reference)
import functools

import jax
import jax.numpy as jnp
from jax import lax
from jax.experimental import pallas as pl
from jax.experimental.pallas import tpu as pltpu
from jax.experimental.pallas import tpu_sc as plsc

F32 = jnp.float32
_HID = 128
_B = 4
_NM = 10000
_N = _B * _NM
_EI_PB = 160000
_EI_PAD = 163840
_EE_PB = 40000
_EE_PAD = 40960
_C = 128
_SC_R = 10240
_DUMP = 10000
_BM_NODE = 2000
_BM_EDGE = 2560

_PREC = None


def _dot(a, b):
    return jnp.dot(a, b, preferred_element_type=F32, precision=_PREC)


def _posmul(pos2, wp):
    return pos2[:, 0:1] * wp[0:1, :] + pos2[:, 1:2] * wp[1:2, :]


def _tc_call(body, row_args, const_args, n_out, bm):
    n = row_args[0].shape[0]
    grid = (n // bm,)
    in_specs = (
        [pl.BlockSpec((bm, a.shape[1]), lambda i: (i, 0)) for a in row_args]
        + [pl.BlockSpec(a.shape, lambda i, nd=a.ndim: (0,) * nd)
           for a in const_args]
    )
    out_shape = [jax.ShapeDtypeStruct((n, _HID), F32) for _ in range(n_out)]
    out_specs = [pl.BlockSpec((bm, _HID), lambda i: (i, 0))
                 for _ in range(n_out)]
    res = pl.pallas_call(
        body,
        grid=grid,
        in_specs=in_specs,
        out_shape=out_shape,
        out_specs=out_specs,
    )(*row_args, *const_args)
    return res if n_out > 1 else res[0]


def _embed_body(u, pos, w1u, w1p, b1, w2, b2, o):
    h = _dot(u[...], w1u[...]) + _posmul(pos[...], w1p[...]) + b1[...]
    h = jnp.tanh(h)
    o[...] = jnp.tanh(_dot(h, w2[...]) + b2[...])


def _prep_int_body(x, pos, ws, wd, wp, b1, s_o, d_o):
    pw = _posmul(pos[...], wp[...])
    s_o[...] = _dot(x[...], ws[...]) - pw
    d_o[...] = _dot(x[...], wd[...]) + pw + b1[...]


def _prep_ext_body(exf, ex_pos, x, pos, wsx, wd, wp, b1, s_o, d_o):
    s_o[...] = _dot(exf[...], wsx[...]) - _posmul(ex_pos[...], wp[...])
    d_o[...] = _dot(x[...], wd[...]) + _posmul(pos[...], wp[...]) + b1[...]


def _msg_body(h, w2, b2, o):
    o[...] = jnp.tanh(_dot(jnp.tanh(h[...]), w2[...]) + b2[...])


def _upd_int_body(x, agg, u, wa, wb, wc, b1, w2, b2, o):
    h = (_dot(x[...], wa[...]) + _dot(agg[...], wb[...])
         + _dot(u[...], wc[...]) + b1[...])
    o[...] = x[...] + _dot(jnp.tanh(h), w2[...]) + b2[...]


def _upd_ext_body(x, agg, wa, wb, b1, w2, b2, o):
    h = _dot(x[...], wa[...]) + _dot(agg[...], wb[...]) + b1[...]
    o[...] = x[...] + _dot(jnp.tanh(h), w2[...]) + b2[...]


def _out_body(x, w1, b1, w2, b2, o):
    h = jnp.tanh(_dot(x[...], w1[...]) + b1[...])
    o[...] = _dot(h, w2[...]) + b2[...]


def _make_gather(epb, gcsz):
    epw = epb // 8
    n_chunks = epw // gcsz
    half = n_chunks // 2
    assert n_chunks % 2 == 0
    mesh = plsc.VectorSubcoreMesh(core_axis_name="c", subcore_axis_name="s")

    @functools.partial(
        pl.kernel, mesh=mesh,
        out_type=jax.ShapeDtypeStruct((_B * epb, _HID), F32),
        scratch_types=[
            pltpu.VMEM((epw,), jnp.int32),
            pltpu.VMEM((epw,), jnp.int32),
            pltpu.VMEM((2, gcsz, _HID), F32),
            pltpu.VMEM((2, gcsz, _HID), F32),
            pltpu.SemaphoreType.DMA,
            pltpu.SemaphoreType.DMA,
        ],
    )
    def gk(s_hbm, d_hbm, src_hbm, dst_hbm, h_hbm,
           idx_s, idx_d, buf_s, buf_d, sem0, sem1):
        wid = lax.axis_index("s") * 2 + lax.axis_index("c")
        batch = wid // 8
        lane = wid % 8
        shift = batch * _NM
        wbase = batch * epb + lane * epw

        pltpu.sync_copy(src_hbm.at[pl.ds(wbase, epw)], idx_s)
        pltpu.sync_copy(dst_hbm.at[pl.ds(wbase, epw)], idx_d)

        def sh(i, carry):
            sl = pl.ds(i * 16, 16)
            idx_s[sl] = idx_s[sl] + shift
            idx_d[sl] = idx_d[sl] + shift
            return carry

        lax.fori_loop(0, epw // 16, sh, 0)

        sems = (sem0, sem1)

        def issue(k, b):
            pltpu.async_copy(s_hbm.at[idx_s.at[pl.ds(k * gcsz, gcsz)]],
                             buf_s.at[b], sems[b])
            pltpu.async_copy(d_hbm.at[idx_d.at[pl.ds(k * gcsz, gcsz)]],
                             buf_d.at[b], sems[b])

        def drain(b):
            pltpu.make_async_copy(h_hbm.at[pl.ds(0, gcsz)],
                                  buf_s.at[b], sems[b]).wait()
            pltpu.make_async_copy(h_hbm.at[pl.ds(0, gcsz)],
                                  buf_d.at[b], sems[b]).wait()

        def process(k, b):
            bs = buf_s.at[b]
            bd = buf_d.at[b]

            def addrow(r, c2):
                for j in range(_HID // 16):
                    sl = pl.ds(j * 16, 16)
                    bs[r, sl] = bs[r, sl] + bd[r, sl]
                return c2

            lax.fori_loop(0, gcsz, addrow, 0)
            pltpu.sync_copy(bs, h_hbm.at[pl.ds(wbase + k * gcsz, gcsz)])

        issue(0, 0)

        def pair(g, carry):
            k0 = g * 2
            drain(0)
            issue(k0 + 1, 1)
            process(k0, 0)
            drain(1)

            @pl.when(g < half - 1)
            def _nxt():
                issue(k0 + 2, 0)

            process(k0 + 1, 1)
            return carry

        lax.fori_loop(0, half, pair, 0)

    return gk


def _make_scatter(epb, csz):
    epw = epb // 16
    n_chunks = epw // csz
    half = n_chunks // 2
    assert n_chunks % 2 == 0
    mesh = plsc.VectorSubcoreMesh(core_axis_name="c", subcore_axis_name="s")

    @functools.partial(
        pl.kernel, mesh=mesh,
        out_type=jax.ShapeDtypeStruct((_N, _HID), F32),
        scratch_types=[
            pltpu.VMEM((2, csz), jnp.int32),
            pltpu.VMEM((2, csz, _HID), F32),
            pltpu.VMEM_SHARED((_SC_R, _HID), F32),
            pltpu.SemaphoreType.DMA,
            pltpu.SemaphoreType.DMA,
        ],
    )
    def sk(m_hbm, dst_hbm, agg_hbm, idx_v, chunk_v, shared, sem0, sem1):
        ci = lax.axis_index("c")
        s = lax.axis_index("s")
        sems = (sem0, sem1)

        for bj in range(2):
            b = ci * 2 + bj
            wbase = b * epb + s * epw

            zb = chunk_v.at[0]

            def zrow(r, carry):
                for j in range(_HID // 16):
                    zb[r, pl.ds(j * 16, 16)] = jnp.zeros((16,), F32)
                return carry

            lax.fori_loop(0, csz, zrow, 0)
            stripe = _SC_R // 16
            for z in range(stripe // csz):
                pltpu.sync_copy(
                    zb, shared.at[pl.ds(s * stripe + z * csz, csz)])
            plsc.subcore_barrier()

            def issue(k, bb):
                base = wbase + k * csz
                pltpu.async_copy(dst_hbm.at[pl.ds(base, csz)],
                                 idx_v.at[bb], sems[bb])
                pltpu.async_copy(m_hbm.at[pl.ds(base, csz)],
                                 chunk_v.at[bb], sems[bb])

            def drain(bb):
                pltpu.make_async_copy(dst_hbm.at[pl.ds(0, csz)],
                                      idx_v.at[bb], sems[bb]).wait()
                pltpu.make_async_copy(m_hbm.at[pl.ds(0, csz)],
                                      chunk_v.at[bb], sems[bb]).wait()

            def scat(bb):
                pltpu.sync_copy(chunk_v.at[bb], shared.at[idx_v.at[bb]],
                                add=True)

            issue(0, 0)

            def pair(g, carry):
                k0 = g * 2
                drain(0)
                issue(k0 + 1, 1)
                scat(0)
                drain(1)

                @pl.when(g < half - 1)
                def _nxt():
                    issue(k0 + 2, 0)

                scat(1)
                return carry

            lax.fori_loop(0, half, pair, 0)
            plsc.subcore_barrier()
            pltpu.sync_copy(shared.at[pl.ds(s * 624, 624)],
                            agg_hbm.at[pl.ds(b * _NM + s * 624, 624)])

            @pl.when(s == 0)
            def _tail():
                pltpu.sync_copy(shared.at[pl.ds(9984, 16)],
                                agg_hbm.at[pl.ds(b * _NM + 9984, 16)])

            plsc.subcore_barrier()

    return sk


def _b2d(v):
    return v.reshape(1, -1)


def _edge_block(s_tab, d_tab, src, dst_g, dst_s, w2, b2, gather, scatter):
    h = gather(s_tab, d_tab, src, dst_g)
    m = _tc_call(_msg_body, [h], [w2, _b2d(b2)], 1, _BM_EDGE)
    return scatter(m, dst_s)


def kernel(madis_x, madis_lon, madis_lat, ex_lon, ex_lat, ex_x, params,
           edge_index, edge_index_e2m):
    p = params
    _gather_int = _make_gather(_EI_PAD, 80)
    _gather_ext = _make_gather(_EE_PAD, 80)
    _scatter_int = _make_scatter(_EI_PAD, _C)
    _scatter_ext = _make_scatter(_EE_PAD, _C)
    B, Nm, Tm, Fm = madis_x.shape
    u = madis_x.reshape(B * Nm, Tm * Fm)
    pos = jnp.concatenate([madis_lon, madis_lat], axis=2).reshape(B * Nm, 2)
    exf = ex_x.reshape(B * ex_x.shape[1], -1)
    ex_pos = jnp.concatenate([ex_lon, ex_lat], axis=2).reshape(-1, 2)

    padw_i = ((0, 0), (0, _EI_PAD - _EI_PB))
    src_i = jnp.pad(edge_index[:, 0, :], padw_i).reshape(-1)
    dst_i_g = jnp.pad(edge_index[:, 1, :], padw_i).reshape(-1)
    dst_i_s = jnp.pad(edge_index[:, 1, :], padw_i,
                      constant_values=_DUMP).reshape(-1)
    padw = ((0, 0), (0, _EE_PAD - _EE_PB))
    src_e = jnp.pad(edge_index_e2m[:, 0, :], padw).reshape(-1)
    dst_e_g = jnp.pad(edge_index_e2m[:, 1, :], padw).reshape(-1)
    dst_e_s = jnp.pad(edge_index_e2m[:, 1, :], padw,
                      constant_values=_DUMP).reshape(-1)

    in_x = _tc_call(
        _embed_body, [u, pos],
        [p["emb_W1"][:128], p["emb_W1"][128:130], _b2d(p["emb_b1"]),
         p["emb_W2"], _b2d(p["emb_b2"])], 1, _BM_NODE)

    for nm in ["ex1"] + ["in%d" % i for i in range(4)] + ["ex2"]:
        ew1 = p[nm + "_eW1"]
        if nm.startswith("ex"):
            s_tab, d_tab = _tc_call(
                _prep_ext_body, [exf, ex_pos, in_x, pos],
                [ew1[:64], ew1[64:192], ew1[192:194], _b2d(p[nm + "_eb1"])],
                2, _BM_NODE)
            agg = _edge_block(s_tab, d_tab, src_e, dst_e_g, dst_e_s,
                              p[nm + "_eW2"], p[nm + "_eb2"],
                              _gather_ext, _scatter_ext)
            in_x = _tc_call(
                _upd_ext_body, [in_x, agg],
                [p[nm + "_nW1"][:128], p[nm + "_nW1"][128:256],
                 _b2d(p[nm + "_nb1"]), p[nm + "_nW2"], _b2d(p[nm + "_nb2"])],
                1, _BM_NODE)
        else:
            s_tab, d_tab = _tc_call(
                _prep_int_body, [in_x, pos],
                [ew1[:128], ew1[128:256], ew1[256:258],
                 _b2d(p[nm + "_eb1"])], 2, _BM_NODE)
            agg = _edge_block(s_tab, d_tab, src_i, dst_i_g, dst_i_s,
                              p[nm + "_eW2"], p[nm + "_eb2"],
                              _gather_int, _scatter_int)
            in_x = _tc_call(
                _upd_int_body, [in_x, agg, u],
                [p[nm + "_nW1"][:128], p[nm + "_nW1"][128:256],
                 p[nm + "_nW1"][256:384], _b2d(p[nm + "_nb1"]),
                 p[nm + "_nW2"], _b2d(p[nm + "_nb2"])], 1, _BM_NODE)

    w2p = jnp.zeros((_HID, _HID), F32).at[:, :2].set(p["out_W2"])
    b2p = jnp.zeros((_HID,), F32).at[:2].set(p["out_b2"])
    out = _tc_call(_out_body, [in_x],
                   [p["out_W1"], _b2d(p["out_b1"]), w2p, _b2d(b2p)],
                   1, _BM_NODE)
    return out[:, :2].reshape(B, Nm, 2)


# speedup vs baseline: 1.2511x; 1.2511x over previous
import functools

import jax
import jax.numpy as jnp
from jax import lax
from jax.experimental import pallas as pl
from jax.experimental.pallas import tpu as pltpu
from jax.experimental.pallas import tpu_sc as plsc

F32 = jnp.float32
_HID = 128
_B = 4
_NM = 10000
_N = _B * _NM
_EI_PB = 160000
_EI_PAD = 163840
_EE_PB = 40000
_EE_PAD = 40960
_C = 128
_SC_R = 10240
_DUMP = 10000
_BM_NODE = 2000
_BM_EDGE = 2560

_PREC = None


def _dot(a, b):
    return jnp.dot(a, b, preferred_element_type=F32, precision=_PREC)


def _posmul(pos2, wp):
    return pos2[:, 0:1] * wp[0:1, :] + pos2[:, 1:2] * wp[1:2, :]


def _tc_call(body, row_args, const_args, n_out, bm):
    n = row_args[0].shape[0]
    grid = (n // bm,)
    in_specs = (
        [pl.BlockSpec((bm, a.shape[1]), lambda i: (i, 0)) for a in row_args]
        + [pl.BlockSpec(a.shape, lambda i, nd=a.ndim: (0,) * nd)
           for a in const_args]
    )
    out_shape = [jax.ShapeDtypeStruct((n, _HID), F32) for _ in range(n_out)]
    out_specs = [pl.BlockSpec((bm, _HID), lambda i: (i, 0))
                 for _ in range(n_out)]
    res = pl.pallas_call(
        body,
        grid=grid,
        in_specs=in_specs,
        out_shape=out_shape,
        out_specs=out_specs,
    )(*row_args, *const_args)
    return res if n_out > 1 else res[0]


def _embed_body(u, pos, w1u, w1p, b1, w2, b2, o):
    h = _dot(u[...], w1u[...]) + _posmul(pos[...], w1p[...]) + b1[...]
    h = jnp.tanh(h)
    o[...] = jnp.tanh(_dot(h, w2[...]) + b2[...])


def _prep_int_body(x, pos, ws, wd, wp, b1, s_o, d_o):
    pw = _posmul(pos[...], wp[...])
    s_o[...] = _dot(x[...], ws[...]) - pw
    d_o[...] = _dot(x[...], wd[...]) + pw + b1[...]


def _prep_ext_body(exf, ex_pos, x, pos, wsx, wd, wp, b1, s_o, d_o):
    s_o[...] = _dot(exf[...], wsx[...]) - _posmul(ex_pos[...], wp[...])
    d_o[...] = _dot(x[...], wd[...]) + _posmul(pos[...], wp[...]) + b1[...]


def _msg_body(h, w2, b2, o):
    o[...] = jnp.tanh(_dot(jnp.tanh(h[...]), w2[...]) + b2[...])


def _upd_int_body(x, agg, u, wa, wb, wc, b1, w2, b2, o):
    h = (_dot(x[...], wa[...]) + _dot(agg[...], wb[...])
         + _dot(u[...], wc[...]) + b1[...])
    o[...] = x[...] + _dot(jnp.tanh(h), w2[...]) + b2[...]


def _upd_ext_body(x, agg, wa, wb, b1, w2, b2, o):
    h = _dot(x[...], wa[...]) + _dot(agg[...], wb[...]) + b1[...]
    o[...] = x[...] + _dot(jnp.tanh(h), w2[...]) + b2[...]


def _out_body(x, w1, b1, w2, b2, o):
    h = jnp.tanh(_dot(x[...], w1[...]) + b1[...])
    o[...] = _dot(h, w2[...]) + b2[...]


def _make_gather(epb, gcsz):
    epw = epb // 8
    n_chunks = epw // gcsz
    half = n_chunks // 2
    assert n_chunks % 2 == 0
    mesh = plsc.VectorSubcoreMesh(core_axis_name="c", subcore_axis_name="s")

    @functools.partial(
        pl.kernel, mesh=mesh,
        out_type=jax.ShapeDtypeStruct((_B * epb, _HID), F32),
        scratch_types=[
            pltpu.VMEM((epw,), jnp.int32),
            pltpu.VMEM((epw,), jnp.int32),
            pltpu.VMEM((2, gcsz, _HID), F32),
            pltpu.VMEM((2, gcsz, _HID), F32),
            pltpu.SemaphoreType.DMA,
            pltpu.SemaphoreType.DMA,
        ],
    )
    def gk(s_hbm, d_hbm, src_hbm, dst_hbm, h_hbm,
           idx_s, idx_d, buf_s, buf_d, sem0, sem1):
        wid = lax.axis_index("s") * 2 + lax.axis_index("c")
        batch = wid // 8
        lane = wid % 8
        shift = batch * _NM
        wbase = batch * epb + lane * epw

        pltpu.sync_copy(src_hbm.at[pl.ds(wbase, epw)], idx_s)
        pltpu.sync_copy(dst_hbm.at[pl.ds(wbase, epw)], idx_d)

        def sh(i, carry):
            sl = pl.ds(i * 16, 16)
            idx_s[sl] = idx_s[sl] + shift
            idx_d[sl] = idx_d[sl] + shift
            return carry

        lax.fori_loop(0, epw // 16, sh, 0)

        sems = (sem0, sem1)

        def issue(k, b):
            pltpu.async_copy(s_hbm.at[idx_s.at[pl.ds(k * gcsz, gcsz)]],
                             buf_s.at[b], sems[b])
            pltpu.async_copy(d_hbm.at[idx_d.at[pl.ds(k * gcsz, gcsz)]],
                             buf_d.at[b], sems[b])

        def drain(b):
            pltpu.make_async_copy(h_hbm.at[pl.ds(0, gcsz)],
                                  buf_s.at[b], sems[b]).wait()
            pltpu.make_async_copy(h_hbm.at[pl.ds(0, gcsz)],
                                  buf_d.at[b], sems[b]).wait()

        def process(k, b):
            bs = buf_s.at[b]
            bd = buf_d.at[b]

            def addrow(r, c2):
                for j in range(_HID // 16):
                    sl = pl.ds(j * 16, 16)
                    bs[r, sl] = bs[r, sl] + bd[r, sl]
                return c2

            lax.fori_loop(0, gcsz, addrow, 0)
            pltpu.sync_copy(bs, h_hbm.at[pl.ds(wbase + k * gcsz, gcsz)])

        issue(0, 0)

        def pair(g, carry):
            k0 = g * 2
            drain(0)
            issue(k0 + 1, 1)
            process(k0, 0)
            drain(1)

            @pl.when(g < half - 1)
            def _nxt():
                issue(k0 + 2, 0)

            process(k0 + 1, 1)
            return carry

        lax.fori_loop(0, half, pair, 0)

    return gk


def _make_scatter(epb, csz):
    epw = epb // 16
    n_full = epw // csz
    rem = epw % csz
    half = n_full // 2
    assert n_full % 2 == 0 and rem % 8 == 0
    trem = max(rem, 8)
    mesh = plsc.VectorSubcoreMesh(core_axis_name="c", subcore_axis_name="s")

    @functools.partial(
        pl.kernel, mesh=mesh,
        out_type=jax.ShapeDtypeStruct((_N, _HID), F32),
        scratch_types=[
            pltpu.VMEM((2, csz), jnp.int32),
            pltpu.VMEM((2, csz, _HID), F32),
            pltpu.VMEM((trem,), jnp.int32),
            pltpu.VMEM((trem, _HID), F32),
            pltpu.VMEM_SHARED((_SC_R, _HID), F32),
            pltpu.SemaphoreType.DMA,
            pltpu.SemaphoreType.DMA,
        ],
    )
    def sk(m_hbm, dst_hbm, agg_hbm, idx_v, chunk_v, idx_t, chunk_t,
           shared, sem0, sem1):
        ci = lax.axis_index("c")
        s = lax.axis_index("s")
        sems = (sem0, sem1)

        for bj in range(2):
            b = ci * 2 + bj
            wbase = b * epb + s * epw

            zb = chunk_v.at[0]

            def zrow(r, carry):
                for j in range(_HID // 16):
                    zb[r, pl.ds(j * 16, 16)] = jnp.zeros((16,), F32)
                return carry

            lax.fori_loop(0, csz, zrow, 0)
            stripe = _SC_R // 16
            for z in range(stripe // csz):
                pltpu.sync_copy(
                    zb, shared.at[pl.ds(s * stripe + z * csz, csz)])
            plsc.subcore_barrier()

            def issue(k, bb):
                base = wbase + k * csz
                pltpu.async_copy(dst_hbm.at[pl.ds(base, csz)],
                                 idx_v.at[bb], sems[bb])
                pltpu.async_copy(m_hbm.at[pl.ds(base, csz)],
                                 chunk_v.at[bb], sems[bb])

            def drain(bb):
                pltpu.make_async_copy(dst_hbm.at[pl.ds(0, csz)],
                                      idx_v.at[bb], sems[bb]).wait()
                pltpu.make_async_copy(m_hbm.at[pl.ds(0, csz)],
                                      chunk_v.at[bb], sems[bb]).wait()

            def scat(bb):
                pltpu.sync_copy(chunk_v.at[bb], shared.at[idx_v.at[bb]],
                                add=True)

            issue(0, 0)

            def pair(g, carry):
                k0 = g * 2
                drain(0)
                issue(k0 + 1, 1)
                scat(0)
                drain(1)

                @pl.when(g < half - 1)
                def _nxt():
                    issue(k0 + 2, 0)

                scat(1)
                return carry

            lax.fori_loop(0, half, pair, 0)
            if rem:
                tbase = wbase + n_full * csz
                pltpu.sync_copy(dst_hbm.at[pl.ds(tbase, rem)], idx_t)
                pltpu.sync_copy(m_hbm.at[pl.ds(tbase, rem)], chunk_t)
                pltpu.sync_copy(chunk_t, shared.at[idx_t], add=True)
            plsc.subcore_barrier()
            pltpu.sync_copy(shared.at[pl.ds(s * 624, 624)],
                            agg_hbm.at[pl.ds(b * _NM + s * 624, 624)])

            @pl.when(s == 0)
            def _tail():
                pltpu.sync_copy(shared.at[pl.ds(9984, 16)],
                                agg_hbm.at[pl.ds(b * _NM + 9984, 16)])

            plsc.subcore_barrier()

    return sk


def _b2d(v):
    return v.reshape(1, -1)


def _edge_block(s_tab, d_tab, src, dst_g, dst_s, w2, b2, gather, scatter):
    h = gather(s_tab, d_tab, src, dst_g)
    m = _tc_call(_msg_body, [h], [w2, _b2d(b2)], 1, _BM_EDGE)
    return scatter(m, dst_s)


def kernel(madis_x, madis_lon, madis_lat, ex_lon, ex_lat, ex_x, params,
           edge_index, edge_index_e2m):
    p = params
    _gather_int = _make_gather(_EI_PB, 80)
    _gather_ext = _make_gather(_EE_PAD, 80)
    _scatter_int = _make_scatter(_EI_PB, _C)
    _scatter_ext = _make_scatter(_EE_PAD, _C)
    B, Nm, Tm, Fm = madis_x.shape
    u = madis_x.reshape(B * Nm, Tm * Fm)
    pos = jnp.concatenate([madis_lon, madis_lat], axis=2).reshape(B * Nm, 2)
    exf = ex_x.reshape(B * ex_x.shape[1], -1)
    ex_pos = jnp.concatenate([ex_lon, ex_lat], axis=2).reshape(-1, 2)

    src_i = edge_index[:, 0, :].reshape(-1)
    dst_i = edge_index[:, 1, :].reshape(-1)
    padw = ((0, 0), (0, _EE_PAD - _EE_PB))
    src_e = jnp.pad(edge_index_e2m[:, 0, :], padw).reshape(-1)
    dst_e_g = jnp.pad(edge_index_e2m[:, 1, :], padw).reshape(-1)
    dst_e_s = jnp.pad(edge_index_e2m[:, 1, :], padw,
                      constant_values=_DUMP).reshape(-1)

    in_x = _tc_call(
        _embed_body, [u, pos],
        [p["emb_W1"][:128], p["emb_W1"][128:130], _b2d(p["emb_b1"]),
         p["emb_W2"], _b2d(p["emb_b2"])], 1, _BM_NODE)

    for nm in ["ex1"] + ["in%d" % i for i in range(4)] + ["ex2"]:
        ew1 = p[nm + "_eW1"]
        if nm.startswith("ex"):
            s_tab, d_tab = _tc_call(
                _prep_ext_body, [exf, ex_pos, in_x, pos],
                [ew1[:64], ew1[64:192], ew1[192:194], _b2d(p[nm + "_eb1"])],
                2, _BM_NODE)
            agg = _edge_block(s_tab, d_tab, src_e, dst_e_g, dst_e_s,
                              p[nm + "_eW2"], p[nm + "_eb2"],
                              _gather_ext, _scatter_ext)
            in_x = _tc_call(
                _upd_ext_body, [in_x, agg],
                [p[nm + "_nW1"][:128], p[nm + "_nW1"][128:256],
                 _b2d(p[nm + "_nb1"]), p[nm + "_nW2"], _b2d(p[nm + "_nb2"])],
                1, _BM_NODE)
        else:
            s_tab, d_tab = _tc_call(
                _prep_int_body, [in_x, pos],
                [ew1[:128], ew1[128:256], ew1[256:258],
                 _b2d(p[nm + "_eb1"])], 2, _BM_NODE)
            agg = _edge_block(s_tab, d_tab, src_i, dst_i, dst_i,
                              p[nm + "_eW2"], p[nm + "_eb2"],
                              _gather_int, _scatter_int)
            in_x = _tc_call(
                _upd_int_body, [in_x, agg, u],
                [p[nm + "_nW1"][:128], p[nm + "_nW1"][128:256],
                 p[nm + "_nW1"][256:384], _b2d(p[nm + "_nb1"]),
                 p[nm + "_nW2"], _b2d(p[nm + "_nb2"])], 1, _BM_NODE)

    w2p = jnp.zeros((_HID, _HID), F32).at[:, :2].set(p["out_W2"])
    b2p = jnp.zeros((_HID,), F32).at[:2].set(p["out_b2"])
    out = _tc_call(_out_body, [in_x],
                   [p["out_W1"], _b2d(p["out_b1"]), w2p, _b2d(b2p)],
                   1, _BM_NODE)
    return out[:, :2].reshape(B, Nm, 2)


# speedup vs baseline: 1.4167x; 1.1323x over previous
import functools

import jax
import jax.numpy as jnp
from jax import lax
from jax.experimental import pallas as pl
from jax.experimental.pallas import tpu as pltpu
from jax.experimental.pallas import tpu_sc as plsc

F32 = jnp.float32
_HID = 128
_B = 4
_NM = 10000
_N = _B * _NM
_EI_PB = 160000
_EI_PAD = 163840
_EE_PB = 40000
_EE_PAD = 40960
_C = 128
_SC_R = 10240
_DUMP = 10000
_BM_NODE = 2000
_BM_EDGE = 2560

_PREC = None


def _dot(a, b):
    return jnp.dot(a, b, preferred_element_type=F32, precision=_PREC)


def _posmul(pos2, wp):
    return pos2[:, 0:1] * wp[0:1, :] + pos2[:, 1:2] * wp[1:2, :]


def _tc_call(body, row_args, const_args, n_out, bm):
    n = row_args[0].shape[0]
    grid = (n // bm,)
    in_specs = (
        [pl.BlockSpec((bm, a.shape[1]), lambda i: (i, 0)) for a in row_args]
        + [pl.BlockSpec(a.shape, lambda i, nd=a.ndim: (0,) * nd)
           for a in const_args]
    )
    out_shape = [jax.ShapeDtypeStruct((n, _HID), F32) for _ in range(n_out)]
    out_specs = [pl.BlockSpec((bm, _HID), lambda i: (i, 0))
                 for _ in range(n_out)]
    res = pl.pallas_call(
        body,
        grid=grid,
        in_specs=in_specs,
        out_shape=out_shape,
        out_specs=out_specs,
    )(*row_args, *const_args)
    return res if n_out > 1 else res[0]


def _embed_body(u, pos, w1u, w1p, b1, w2, b2, o):
    h = _dot(u[...], w1u[...]) + _posmul(pos[...], w1p[...]) + b1[...]
    h = jnp.tanh(h)
    o[...] = jnp.tanh(_dot(h, w2[...]) + b2[...])


def _prep_int_body(x, pos, ws, wd, wp, b1, s_o, d_o):
    pw = _posmul(pos[...], wp[...])
    s_o[...] = _dot(x[...], ws[...]) - pw
    d_o[...] = _dot(x[...], wd[...]) + pw + b1[...]


def _prep_ext_body(exf, ex_pos, x, pos, wsx, wd, wp, b1, s_o, d_o):
    s_o[...] = _dot(exf[...], wsx[...]) - _posmul(ex_pos[...], wp[...])
    d_o[...] = _dot(x[...], wd[...]) + _posmul(pos[...], wp[...]) + b1[...]


def _msg_body(h, w2, b2, o):
    o[...] = jnp.tanh(_dot(jnp.tanh(h[...]), w2[...]) + b2[...])


def _upd_int_body(x, agg, u, wa, wb, wc, b1, w2, b2, o):
    h = (_dot(x[...], wa[...]) + _dot(agg[...], wb[...])
         + _dot(u[...], wc[...]) + b1[...])
    o[...] = x[...] + _dot(jnp.tanh(h), w2[...]) + b2[...]


def _upd_ext_body(x, agg, wa, wb, b1, w2, b2, o):
    h = _dot(x[...], wa[...]) + _dot(agg[...], wb[...]) + b1[...]
    o[...] = x[...] + _dot(jnp.tanh(h), w2[...]) + b2[...]


def _out_body(x, w1, b1, w2, b2, o):
    h = jnp.tanh(_dot(x[...], w1[...]) + b1[...])
    o[...] = _dot(h, w2[...]) + b2[...]


def _make_gather(epb, gcsz, b0):
    epw = epb // 16
    n_chunks = epw // gcsz
    half = n_chunks // 2
    odd = n_chunks % 2 == 1
    mesh = plsc.VectorSubcoreMesh(core_axis_name="c", subcore_axis_name="s")

    @functools.partial(
        pl.kernel, mesh=mesh,
        out_type=jax.ShapeDtypeStruct((2 * epb, _HID), F32),
        scratch_types=[
            pltpu.VMEM((epw,), jnp.int32),
            pltpu.VMEM((epw,), jnp.int32),
            pltpu.VMEM((2, gcsz, _HID), F32),
            pltpu.VMEM((2, gcsz, _HID), F32),
            pltpu.SemaphoreType.DMA,
            pltpu.SemaphoreType.DMA,
        ],
    )
    def gk(s_hbm, d_hbm, src_hbm, dst_hbm, h_hbm,
           idx_s, idx_d, buf_s, buf_d, sem0, sem1):
        wid = lax.axis_index("s") * 2 + lax.axis_index("c")
        batch = wid // 16
        lane = wid % 16
        shift = (batch + b0) * _NM
        wbase = batch * epb + lane * epw

        pltpu.sync_copy(src_hbm.at[pl.ds(wbase, epw)], idx_s)
        pltpu.sync_copy(dst_hbm.at[pl.ds(wbase, epw)], idx_d)

        def sh(i, carry):
            sl = pl.ds(i * 16, 16)
            idx_s[sl] = idx_s[sl] + shift
            idx_d[sl] = idx_d[sl] + shift
            return carry

        lax.fori_loop(0, epw // 16, sh, 0)

        sems = (sem0, sem1)

        def issue(k, b):
            pltpu.async_copy(s_hbm.at[idx_s.at[pl.ds(k * gcsz, gcsz)]],
                             buf_s.at[b], sems[b])
            pltpu.async_copy(d_hbm.at[idx_d.at[pl.ds(k * gcsz, gcsz)]],
                             buf_d.at[b], sems[b])

        def drain(b):
            pltpu.make_async_copy(h_hbm.at[pl.ds(0, gcsz)],
                                  buf_s.at[b], sems[b]).wait()
            pltpu.make_async_copy(h_hbm.at[pl.ds(0, gcsz)],
                                  buf_d.at[b], sems[b]).wait()

        def process(k, b):
            bs = buf_s.at[b]
            bd = buf_d.at[b]

            def addrow(r, c2):
                for j in range(_HID // 16):
                    sl = pl.ds(j * 16, 16)
                    bs[r, sl] = bs[r, sl] + bd[r, sl]
                return c2

            lax.fori_loop(0, gcsz, addrow, 0)
            pltpu.sync_copy(bs, h_hbm.at[pl.ds(wbase + k * gcsz, gcsz)])

        issue(0, 0)

        def pair(g, carry):
            k0 = g * 2
            drain(0)
            issue(k0 + 1, 1)
            process(k0, 0)
            drain(1)

            if odd:
                issue(k0 + 2, 0)
            else:
                @pl.when(g < half - 1)
                def _nxt():
                    issue(k0 + 2, 0)

            process(k0 + 1, 1)
            return carry

        lax.fori_loop(0, half, pair, 0)
        if odd:
            drain(0)
            process(n_chunks - 1, 0)

    return gk


def _make_scatter(epb, csz):
    epw = epb // 16
    n_full = epw // csz
    rem = epw % csz
    half = n_full // 2
    assert n_full % 2 == 0 and rem % 8 == 0
    trem = max(rem, 8)
    mesh = plsc.VectorSubcoreMesh(core_axis_name="c", subcore_axis_name="s")

    @functools.partial(
        pl.kernel, mesh=mesh,
        out_type=jax.ShapeDtypeStruct((2 * _NM, _HID), F32),
        scratch_types=[
            pltpu.VMEM((2, csz), jnp.int32),
            pltpu.VMEM((2, csz, _HID), F32),
            pltpu.VMEM((trem,), jnp.int32),
            pltpu.VMEM((trem, _HID), F32),
            pltpu.VMEM_SHARED((_SC_R, _HID), F32),
            pltpu.SemaphoreType.DMA,
            pltpu.SemaphoreType.DMA,
        ],
    )
    def sk(m_hbm, dst_hbm, agg_hbm, idx_v, chunk_v, idx_t, chunk_t,
           shared, sem0, sem1):
        b = lax.axis_index("c")
        s = lax.axis_index("s")
        sems = (sem0, sem1)
        wbase = b * epb + s * epw

        zb = chunk_v.at[0]

        def zrow(r, carry):
            for j in range(_HID // 16):
                zb[r, pl.ds(j * 16, 16)] = jnp.zeros((16,), F32)
            return carry

        lax.fori_loop(0, csz, zrow, 0)
        stripe = _SC_R // 16
        for z in range(stripe // csz):
            pltpu.sync_copy(
                zb, shared.at[pl.ds(s * stripe + z * csz, csz)])
        plsc.subcore_barrier()

        def issue(k, bb):
            base = wbase + k * csz
            pltpu.async_copy(dst_hbm.at[pl.ds(base, csz)],
                             idx_v.at[bb], sems[bb])
            pltpu.async_copy(m_hbm.at[pl.ds(base, csz)],
                             chunk_v.at[bb], sems[bb])

        def drain(bb):
            pltpu.make_async_copy(dst_hbm.at[pl.ds(0, csz)],
                                  idx_v.at[bb], sems[bb]).wait()
            pltpu.make_async_copy(m_hbm.at[pl.ds(0, csz)],
                                  chunk_v.at[bb], sems[bb]).wait()

        def scat(bb):
            pltpu.sync_copy(chunk_v.at[bb], shared.at[idx_v.at[bb]],
                            add=True)

        issue(0, 0)

        def pair(g, carry):
            k0 = g * 2
            drain(0)
            issue(k0 + 1, 1)
            scat(0)
            drain(1)

            @pl.when(g < half - 1)
            def _nxt():
                issue(k0 + 2, 0)

            scat(1)
            return carry

        lax.fori_loop(0, half, pair, 0)
        if rem:
            tbase = wbase + n_full * csz
            pltpu.sync_copy(dst_hbm.at[pl.ds(tbase, rem)], idx_t)
            pltpu.sync_copy(m_hbm.at[pl.ds(tbase, rem)], chunk_t)
            pltpu.sync_copy(chunk_t, shared.at[idx_t], add=True)
        plsc.subcore_barrier()
        pltpu.sync_copy(shared.at[pl.ds(s * 624, 624)],
                        agg_hbm.at[pl.ds(b * _NM + s * 624, 624)])

        @pl.when(s == 0)
        def _tail():
            pltpu.sync_copy(shared.at[pl.ds(9984, 16)],
                            agg_hbm.at[pl.ds(b * _NM + 9984, 16)])

    return sk


def _b2d(v):
    return v.reshape(1, -1)


def _edge_block(s_tab, d_tab, src, dst_g, dst_s, w2, b2, g_pair, scatter,
                epb):
    half = 2 * epb
    aggs = []
    for i, g in enumerate(g_pair):
        sl = slice(i * half, (i + 1) * half)
        h = g(s_tab, d_tab, src[sl], dst_g[sl])
        m = _tc_call(_msg_body, [h], [w2, _b2d(b2)], 1, _BM_EDGE)
        aggs.append(scatter(m, dst_s[sl]))
    return jnp.concatenate(aggs, axis=0)


def kernel(madis_x, madis_lon, madis_lat, ex_lon, ex_lat, ex_x, params,
           edge_index, edge_index_e2m):
    p = params
    _gather_int = (_make_gather(_EI_PB, 80, 0), _make_gather(_EI_PB, 80, 2))
    _gather_ext = (_make_gather(_EE_PAD, 80, 0),
                   _make_gather(_EE_PAD, 80, 2))
    _scatter_int = _make_scatter(_EI_PB, _C)
    _scatter_ext = _make_scatter(_EE_PAD, _C)
    B, Nm, Tm, Fm = madis_x.shape
    u = madis_x.reshape(B * Nm, Tm * Fm)
    pos = jnp.concatenate([madis_lon, madis_lat], axis=2).reshape(B * Nm, 2)
    exf = ex_x.reshape(B * ex_x.shape[1], -1)
    ex_pos = jnp.concatenate([ex_lon, ex_lat], axis=2).reshape(-1, 2)

    src_i = edge_index[:, 0, :].reshape(-1)
    dst_i = edge_index[:, 1, :].reshape(-1)
    padw = ((0, 0), (0, _EE_PAD - _EE_PB))
    src_e = jnp.pad(edge_index_e2m[:, 0, :], padw).reshape(-1)
    dst_e_g = jnp.pad(edge_index_e2m[:, 1, :], padw).reshape(-1)
    dst_e_s = jnp.pad(edge_index_e2m[:, 1, :], padw,
                      constant_values=_DUMP).reshape(-1)

    in_x = _tc_call(
        _embed_body, [u, pos],
        [p["emb_W1"][:128], p["emb_W1"][128:130], _b2d(p["emb_b1"]),
         p["emb_W2"], _b2d(p["emb_b2"])], 1, _BM_NODE)

    for nm in ["ex1"] + ["in%d" % i for i in range(4)] + ["ex2"]:
        ew1 = p[nm + "_eW1"]
        if nm.startswith("ex"):
            s_tab, d_tab = _tc_call(
                _prep_ext_body, [exf, ex_pos, in_x, pos],
                [ew1[:64], ew1[64:192], ew1[192:194], _b2d(p[nm + "_eb1"])],
                2, _BM_NODE)
            agg = _edge_block(s_tab, d_tab, src_e, dst_e_g, dst_e_s,
                              p[nm + "_eW2"], p[nm + "_eb2"],
                              _gather_ext, _scatter_ext, _EE_PAD)
            in_x = _tc_call(
                _upd_ext_body, [in_x, agg],
                [p[nm + "_nW1"][:128], p[nm + "_nW1"][128:256],
                 _b2d(p[nm + "_nb1"]), p[nm + "_nW2"], _b2d(p[nm + "_nb2"])],
                1, _BM_NODE)
        else:
            s_tab, d_tab = _tc_call(
                _prep_int_body, [in_x, pos],
                [ew1[:128], ew1[128:256], ew1[256:258],
                 _b2d(p[nm + "_eb1"])], 2, _BM_NODE)
            agg = _edge_block(s_tab, d_tab, src_i, dst_i, dst_i,
                              p[nm + "_eW2"], p[nm + "_eb2"],
                              _gather_int, _scatter_int, _EI_PB)
            in_x = _tc_call(
                _upd_int_body, [in_x, agg, u],
                [p[nm + "_nW1"][:128], p[nm + "_nW1"][128:256],
                 p[nm + "_nW1"][256:384], _b2d(p[nm + "_nb1"]),
                 p[nm + "_nW2"], _b2d(p[nm + "_nb2"])], 1, _BM_NODE)

    w2p = jnp.zeros((_HID, _HID), F32).at[:, :2].set(p["out_W2"])
    b2p = jnp.zeros((_HID,), F32).at[:2].set(p["out_b2"])
    out = _tc_call(_out_body, [in_x],
                   [p["out_W1"], _b2d(p["out_b1"]), w2p, _b2d(b2p)],
                   1, _BM_NODE)
    return out[:, :2].reshape(B, Nm, 2)


# speedup vs baseline: 1.5125x; 1.0677x over previous
import functools

import jax
import jax.numpy as jnp
from jax import lax
from jax.experimental import pallas as pl
from jax.experimental.pallas import tpu as pltpu
from jax.experimental.pallas import tpu_sc as plsc

F32 = jnp.float32
_HID = 128
_B = 4
_NM = 10000
_N = _B * _NM
_EI_PB = 160000
_EI_PAD = 163840
_EE_PB = 40000
_EE_PAD = 40960
_C = 128
_SC_R = 10240
_DUMP = 10000
_BM_NODE = 2000
_BM_EDGE = 2560

_PREC = None


def _dot(a, b):
    return jnp.dot(a, b, preferred_element_type=F32, precision=_PREC)


def _posmul(pos2, wp):
    return pos2[:, 0:1] * wp[0:1, :] + pos2[:, 1:2] * wp[1:2, :]


def _tc_call(body, row_args, const_args, n_out, bm):
    n = row_args[0].shape[0]
    grid = (n // bm,)
    in_specs = (
        [pl.BlockSpec((bm, a.shape[1]), lambda i: (i, 0)) for a in row_args]
        + [pl.BlockSpec(a.shape, lambda i, nd=a.ndim: (0,) * nd)
           for a in const_args]
    )
    out_shape = [jax.ShapeDtypeStruct((n, _HID), F32) for _ in range(n_out)]
    out_specs = [pl.BlockSpec((bm, _HID), lambda i: (i, 0))
                 for _ in range(n_out)]
    res = pl.pallas_call(
        body,
        grid=grid,
        in_specs=in_specs,
        out_shape=out_shape,
        out_specs=out_specs,
    )(*row_args, *const_args)
    return res if n_out > 1 else res[0]


def _embed_body(u, pos, w1u, w1p, b1, w2, b2, o):
    h = _dot(u[...], w1u[...]) + _posmul(pos[...], w1p[...]) + b1[...]
    h = jnp.tanh(h)
    o[...] = jnp.tanh(_dot(h, w2[...]) + b2[...])


def _prep_int_body(x, pos, ws, wd, wp, b1, s_o, d_o):
    pw = _posmul(pos[...], wp[...])
    s_o[...] = _dot(x[...], ws[...]) - pw
    d_o[...] = _dot(x[...], wd[...]) + pw + b1[...]


def _prep_ext_body(exf, ex_pos, x, pos, wsx, wd, wp, b1, s_o, d_o):
    s_o[...] = _dot(exf[...], wsx[...]) - _posmul(ex_pos[...], wp[...])
    d_o[...] = _dot(x[...], wd[...]) + _posmul(pos[...], wp[...]) + b1[...]


def _msg_body(h, w2, b2, o):
    o[...] = jnp.tanh(_dot(jnp.tanh(h[...]), w2[...]) + b2[...])


def _upd_int_body(x, agg, u, wa, wb, wc, b1, w2, b2, o):
    h = (_dot(x[...], wa[...]) + _dot(agg[...], wb[...])
         + _dot(u[...], wc[...]) + b1[...])
    o[...] = x[...] + _dot(jnp.tanh(h), w2[...]) + b2[...]


def _upd_ext_body(x, agg, wa, wb, b1, w2, b2, o):
    h = _dot(x[...], wa[...]) + _dot(agg[...], wb[...]) + b1[...]
    o[...] = x[...] + _dot(jnp.tanh(h), w2[...]) + b2[...]


def _out_body(x, w1, b1, w2, b2, o):
    h = jnp.tanh(_dot(x[...], w1[...]) + b1[...])
    o[...] = _dot(h, w2[...]) + b2[...]


def _make_gather(epb, gcsz, b0):
    epw = epb // 16
    n_chunks = epw // gcsz
    nd = 4
    mesh = plsc.VectorSubcoreMesh(core_axis_name="c", subcore_axis_name="s")

    @functools.partial(
        pl.kernel, mesh=mesh,
        out_type=jax.ShapeDtypeStruct((2 * epb, _HID), F32),
        scratch_types=[
            pltpu.VMEM((epw,), jnp.int32),
            pltpu.VMEM((epw,), jnp.int32),
            pltpu.VMEM((nd, gcsz, _HID), F32),
            pltpu.VMEM((nd, gcsz, _HID), F32),
            pltpu.SemaphoreType.DMA,
            pltpu.SemaphoreType.DMA,
            pltpu.SemaphoreType.DMA,
            pltpu.SemaphoreType.DMA,
        ],
    )
    def gk(s_hbm, d_hbm, src_hbm, dst_hbm, h_hbm,
           idx_s, idx_d, buf_s, buf_d, sem0, sem1, sem2, sem3):
        wid = lax.axis_index("s") * 2 + lax.axis_index("c")
        batch = wid // 16
        lane = wid % 16
        shift = (batch + b0) * _NM
        wbase = batch * epb + lane * epw

        pltpu.sync_copy(src_hbm.at[pl.ds(wbase, epw)], idx_s)
        pltpu.sync_copy(dst_hbm.at[pl.ds(wbase, epw)], idx_d)

        def sh(i, carry):
            sl = pl.ds(i * 16, 16)
            idx_s[sl] = idx_s[sl] + shift
            idx_d[sl] = idx_d[sl] + shift
            return carry

        lax.fori_loop(0, epw // 16, sh, 0)

        sems = (sem0, sem1, sem2, sem3)

        def issue(k, b):
            pltpu.async_copy(s_hbm.at[idx_s.at[pl.ds(k * gcsz, gcsz)]],
                             buf_s.at[b], sems[b])
            pltpu.async_copy(d_hbm.at[idx_d.at[pl.ds(k * gcsz, gcsz)]],
                             buf_d.at[b], sems[b])

        def drain(b):
            pltpu.make_async_copy(h_hbm.at[pl.ds(0, gcsz)],
                                  buf_s.at[b], sems[b]).wait()
            pltpu.make_async_copy(h_hbm.at[pl.ds(0, gcsz)],
                                  buf_d.at[b], sems[b]).wait()

        def process(k, b):
            bs = buf_s.at[b]
            bd = buf_d.at[b]

            def addrow(r, c2):
                for j in range(_HID // 16):
                    sl = pl.ds(j * 16, 16)
                    bs[r, sl] = bs[r, sl] + bd[r, sl]
                return c2

            lax.fori_loop(0, gcsz, addrow, 0)
            pltpu.sync_copy(bs, h_hbm.at[pl.ds(wbase + k * gcsz, gcsz)])

        for i in range(min(nd - 1, n_chunks)):
            issue(i, i % nd)

        def quad(q, carry):
            for b in range(nd):
                k = q * nd + b
                drain(b)

                @pl.when(k + nd - 1 < n_chunks)
                def _nxt():
                    issue(k + nd - 1, (b + nd - 1) % nd)

                process(k, b)
            return carry

        lax.fori_loop(0, n_chunks // nd, quad, 0)
        for k in range(n_chunks - n_chunks % nd, n_chunks):
            b = k % nd
            drain(b)
            if k + nd - 1 < n_chunks:
                issue(k + nd - 1, (k + nd - 1) % nd)
            process(k, b)

    return gk


def _make_scatter(epb, csz):
    epw = epb // 16
    n_full = epw // csz
    rem = epw % csz
    half = n_full // 2
    assert n_full % 2 == 0 and rem % 8 == 0
    trem = max(rem, 8)
    mesh = plsc.VectorSubcoreMesh(core_axis_name="c", subcore_axis_name="s")

    @functools.partial(
        pl.kernel, mesh=mesh,
        out_type=jax.ShapeDtypeStruct((2 * _NM, _HID), F32),
        scratch_types=[
            pltpu.VMEM((2, csz), jnp.int32),
            pltpu.VMEM((2, csz, _HID), F32),
            pltpu.VMEM((trem,), jnp.int32),
            pltpu.VMEM((trem, _HID), F32),
            pltpu.VMEM_SHARED((_SC_R, _HID), F32),
            pltpu.SemaphoreType.DMA,
            pltpu.SemaphoreType.DMA,
        ],
    )
    def sk(m_hbm, dst_hbm, agg_hbm, idx_v, chunk_v, idx_t, chunk_t,
           shared, sem0, sem1):
        b = lax.axis_index("c")
        s = lax.axis_index("s")
        sems = (sem0, sem1)
        wbase = b * epb + s * epw

        zb = chunk_v.at[0]

        def zrow(r, carry):
            for j in range(_HID // 16):
                zb[r, pl.ds(j * 16, 16)] = jnp.zeros((16,), F32)
            return carry

        lax.fori_loop(0, csz, zrow, 0)
        stripe = _SC_R // 16
        for z in range(stripe // csz):
            pltpu.sync_copy(
                zb, shared.at[pl.ds(s * stripe + z * csz, csz)])
        plsc.subcore_barrier()

        def issue(k, bb):
            base = wbase + k * csz
            pltpu.async_copy(dst_hbm.at[pl.ds(base, csz)],
                             idx_v.at[bb], sems[bb])
            pltpu.async_copy(m_hbm.at[pl.ds(base, csz)],
                             chunk_v.at[bb], sems[bb])

        def drain(bb):
            pltpu.make_async_copy(dst_hbm.at[pl.ds(0, csz)],
                                  idx_v.at[bb], sems[bb]).wait()
            pltpu.make_async_copy(m_hbm.at[pl.ds(0, csz)],
                                  chunk_v.at[bb], sems[bb]).wait()

        def scat(bb):
            pltpu.sync_copy(chunk_v.at[bb], shared.at[idx_v.at[bb]],
                            add=True)

        issue(0, 0)

        def pair(g, carry):
            k0 = g * 2
            drain(0)
            issue(k0 + 1, 1)
            scat(0)
            drain(1)

            @pl.when(g < half - 1)
            def _nxt():
                issue(k0 + 2, 0)

            scat(1)
            return carry

        lax.fori_loop(0, half, pair, 0)
        if rem:
            tbase = wbase + n_full * csz
            pltpu.sync_copy(dst_hbm.at[pl.ds(tbase, rem)], idx_t)
            pltpu.sync_copy(m_hbm.at[pl.ds(tbase, rem)], chunk_t)
            pltpu.sync_copy(chunk_t, shared.at[idx_t], add=True)
        plsc.subcore_barrier()
        pltpu.sync_copy(shared.at[pl.ds(s * 624, 624)],
                        agg_hbm.at[pl.ds(b * _NM + s * 624, 624)])

        @pl.when(s == 0)
        def _tail():
            pltpu.sync_copy(shared.at[pl.ds(9984, 16)],
                            agg_hbm.at[pl.ds(b * _NM + 9984, 16)])

    return sk


def _b2d(v):
    return v.reshape(1, -1)


def _edge_block(s_tab, d_tab, src, dst_g, dst_s, w2, b2, g_pair, scatter,
                epb):
    half = 2 * epb
    aggs = []
    for i, g in enumerate(g_pair):
        sl = slice(i * half, (i + 1) * half)
        h = g(s_tab, d_tab, src[sl], dst_g[sl])
        m = _tc_call(_msg_body, [h], [w2, _b2d(b2)], 1, _BM_EDGE)
        aggs.append(scatter(m, dst_s[sl]))
    return jnp.concatenate(aggs, axis=0)


def kernel(madis_x, madis_lon, madis_lat, ex_lon, ex_lat, ex_x, params,
           edge_index, edge_index_e2m):
    p = params
    _gather_int = (_make_gather(_EI_PB, 80, 0), _make_gather(_EI_PB, 80, 2))
    _gather_ext = (_make_gather(_EE_PAD, 80, 0),
                   _make_gather(_EE_PAD, 80, 2))
    _scatter_int = _make_scatter(_EI_PB, _C)
    _scatter_ext = _make_scatter(_EE_PAD, _C)
    B, Nm, Tm, Fm = madis_x.shape
    u = madis_x.reshape(B * Nm, Tm * Fm)
    pos = jnp.concatenate([madis_lon, madis_lat], axis=2).reshape(B * Nm, 2)
    exf = ex_x.reshape(B * ex_x.shape[1], -1)
    ex_pos = jnp.concatenate([ex_lon, ex_lat], axis=2).reshape(-1, 2)

    src_i = edge_index[:, 0, :].reshape(-1)
    dst_i = edge_index[:, 1, :].reshape(-1)
    padw = ((0, 0), (0, _EE_PAD - _EE_PB))
    src_e = jnp.pad(edge_index_e2m[:, 0, :], padw).reshape(-1)
    dst_e_g = jnp.pad(edge_index_e2m[:, 1, :], padw).reshape(-1)
    dst_e_s = jnp.pad(edge_index_e2m[:, 1, :], padw,
                      constant_values=_DUMP).reshape(-1)

    in_x = _tc_call(
        _embed_body, [u, pos],
        [p["emb_W1"][:128], p["emb_W1"][128:130], _b2d(p["emb_b1"]),
         p["emb_W2"], _b2d(p["emb_b2"])], 1, _BM_NODE)

    for nm in ["ex1"] + ["in%d" % i for i in range(4)] + ["ex2"]:
        ew1 = p[nm + "_eW1"]
        if nm.startswith("ex"):
            s_tab, d_tab = _tc_call(
                _prep_ext_body, [exf, ex_pos, in_x, pos],
                [ew1[:64], ew1[64:192], ew1[192:194], _b2d(p[nm + "_eb1"])],
                2, _BM_NODE)
            agg = _edge_block(s_tab, d_tab, src_e, dst_e_g, dst_e_s,
                              p[nm + "_eW2"], p[nm + "_eb2"],
                              _gather_ext, _scatter_ext, _EE_PAD)
            in_x = _tc_call(
                _upd_ext_body, [in_x, agg],
                [p[nm + "_nW1"][:128], p[nm + "_nW1"][128:256],
                 _b2d(p[nm + "_nb1"]), p[nm + "_nW2"], _b2d(p[nm + "_nb2"])],
                1, _BM_NODE)
        else:
            s_tab, d_tab = _tc_call(
                _prep_int_body, [in_x, pos],
                [ew1[:128], ew1[128:256], ew1[256:258],
                 _b2d(p[nm + "_eb1"])], 2, _BM_NODE)
            agg = _edge_block(s_tab, d_tab, src_i, dst_i, dst_i,
                              p[nm + "_eW2"], p[nm + "_eb2"],
                              _gather_int, _scatter_int, _EI_PB)
            in_x = _tc_call(
                _upd_int_body, [in_x, agg, u],
                [p[nm + "_nW1"][:128], p[nm + "_nW1"][128:256],
                 p[nm + "_nW1"][256:384], _b2d(p[nm + "_nb1"]),
                 p[nm + "_nW2"], _b2d(p[nm + "_nb2"])], 1, _BM_NODE)

    w2p = jnp.zeros((_HID, _HID), F32).at[:, :2].set(p["out_W2"])
    b2p = jnp.zeros((_HID,), F32).at[:2].set(p["out_b2"])
    out = _tc_call(_out_body, [in_x],
                   [p["out_W1"], _b2d(p["out_b1"]), w2p, _b2d(b2p)],
                   1, _BM_NODE)
    return out[:, :2].reshape(B, Nm, 2)


# speedup vs baseline: 1.5523x; 1.0263x over previous
import functools

import jax
import jax.numpy as jnp
from jax import lax
from jax.experimental import pallas as pl
from jax.experimental.pallas import tpu as pltpu
from jax.experimental.pallas import tpu_sc as plsc

F32 = jnp.float32
_HID = 128
_B = 4
_NM = 10000
_N = _B * _NM
_EI_PB = 160000
_EI_PAD = 163840
_EE_PB = 40000
_EE_PAD = 40960
_C = 128
_SC_R = 10240
_DUMP = 10000
_BM_NODE = 2000
_BM_EDGE = 2560

_PREC = None


def _dot(a, b):
    return jnp.dot(a, b, preferred_element_type=F32, precision=_PREC)


def _posmul(pos2, wp):
    return pos2[:, 0:1] * wp[0:1, :] + pos2[:, 1:2] * wp[1:2, :]


def _tc_call(body, row_args, const_args, n_out, bm):
    n = row_args[0].shape[0]
    grid = (n // bm,)
    in_specs = (
        [pl.BlockSpec((bm, a.shape[1]), lambda i: (i, 0)) for a in row_args]
        + [pl.BlockSpec(a.shape, lambda i, nd=a.ndim: (0,) * nd)
           for a in const_args]
    )
    out_shape = [jax.ShapeDtypeStruct((n, _HID), F32) for _ in range(n_out)]
    out_specs = [pl.BlockSpec((bm, _HID), lambda i: (i, 0))
                 for _ in range(n_out)]
    res = pl.pallas_call(
        body,
        grid=grid,
        in_specs=in_specs,
        out_shape=out_shape,
        out_specs=out_specs,
    )(*row_args, *const_args)
    return res if n_out > 1 else res[0]


def _embed_body(u, pos, w1u, w1p, b1, w2, b2, o):
    h = _dot(u[...], w1u[...]) + _posmul(pos[...], w1p[...]) + b1[...]
    h = jnp.tanh(h)
    o[...] = jnp.tanh(_dot(h, w2[...]) + b2[...])


def _prep_int_body(x, pos, ws, wd, wp, b1, s_o, d_o):
    pw = _posmul(pos[...], wp[...])
    s_o[...] = _dot(x[...], ws[...]) - pw
    d_o[...] = _dot(x[...], wd[...]) + pw + b1[...]


def _prep_ext_body(exf, ex_pos, x, pos, wsx, wd, wp, b1, s_o, d_o):
    s_o[...] = _dot(exf[...], wsx[...]) - _posmul(ex_pos[...], wp[...])
    d_o[...] = _dot(x[...], wd[...]) + _posmul(pos[...], wp[...]) + b1[...]


def _msg_body(h, w2, b2, o):
    o[...] = jnp.tanh(_dot(jnp.tanh(h[...]), w2[...]) + b2[...])


def _upd_int_body(x, agg, u, wa, wb, wc, b1, w2, b2, o):
    h = (_dot(x[...], wa[...]) + _dot(agg[...], wb[...])
         + _dot(u[...], wc[...]) + b1[...])
    o[...] = x[...] + _dot(jnp.tanh(h), w2[...]) + b2[...]


def _upd_ext_body(x, agg, wa, wb, b1, w2, b2, o):
    h = _dot(x[...], wa[...]) + _dot(agg[...], wb[...]) + b1[...]
    o[...] = x[...] + _dot(jnp.tanh(h), w2[...]) + b2[...]


def _out_body(x, w1, b1, w2, b2, o):
    h = jnp.tanh(_dot(x[...], w1[...]) + b1[...])
    o[...] = _dot(h, w2[...]) + b2[...]


def _make_gather(epb, gcsz, b0):
    epw = epb // 16
    n_chunks = epw // gcsz
    nd = 4
    mesh = plsc.VectorSubcoreMesh(core_axis_name="c", subcore_axis_name="s")

    @functools.partial(
        pl.kernel, mesh=mesh,
        out_type=jax.ShapeDtypeStruct((2 * epb, _HID), F32),
        scratch_types=[
            pltpu.VMEM((epw,), jnp.int32),
            pltpu.VMEM((epw,), jnp.int32),
            pltpu.VMEM((nd, gcsz, _HID), F32),
            pltpu.VMEM((nd, gcsz, _HID), F32),
            pltpu.SemaphoreType.DMA,
            pltpu.SemaphoreType.DMA,
            pltpu.SemaphoreType.DMA,
            pltpu.SemaphoreType.DMA,
        ],
    )
    def gk(s_hbm, d_hbm, src_hbm, dst_hbm, h_hbm,
           idx_s, idx_d, buf_s, buf_d, sem0, sem1, sem2, sem3):
        wid = lax.axis_index("s") * 2 + lax.axis_index("c")
        batch = wid // 16
        lane = wid % 16
        shift = (batch + b0) * _NM
        wbase = batch * epb + lane * epw

        pltpu.sync_copy(src_hbm.at[pl.ds(wbase, epw)], idx_s)
        pltpu.sync_copy(dst_hbm.at[pl.ds(wbase, epw)], idx_d)

        def sh(i, carry):
            sl = pl.ds(i * 16, 16)
            idx_s[sl] = idx_s[sl] + shift
            idx_d[sl] = idx_d[sl] + shift
            return carry

        lax.fori_loop(0, epw // 16, sh, 0)

        sems = (sem0, sem1, sem2, sem3)

        def issue(k, b):
            pltpu.async_copy(s_hbm.at[idx_s.at[pl.ds(k * gcsz, gcsz)]],
                             buf_s.at[b], sems[b])
            pltpu.async_copy(d_hbm.at[idx_d.at[pl.ds(k * gcsz, gcsz)]],
                             buf_d.at[b], sems[b])

        def drain(b):
            pltpu.make_async_copy(h_hbm.at[pl.ds(0, gcsz)],
                                  buf_s.at[b], sems[b]).wait()
            pltpu.make_async_copy(h_hbm.at[pl.ds(0, gcsz)],
                                  buf_d.at[b], sems[b]).wait()

        def process(k, b):
            bs = buf_s.at[b]
            bd = buf_d.at[b]

            def addrow(r, c2):
                for j in range(_HID // 16):
                    sl = pl.ds(j * 16, 16)
                    bs[r, sl] = bs[r, sl] + bd[r, sl]
                return c2

            lax.fori_loop(0, gcsz, addrow, 0)
            pltpu.sync_copy(bs, h_hbm.at[pl.ds(wbase + k * gcsz, gcsz)])

        for i in range(min(nd - 1, n_chunks)):
            issue(i, i % nd)

        def quad(q, carry):
            for b in range(nd):
                k = q * nd + b
                drain(b)

                @pl.when(k + nd - 1 < n_chunks)
                def _nxt():
                    issue(k + nd - 1, (b + nd - 1) % nd)

                process(k, b)
            return carry

        lax.fori_loop(0, n_chunks // nd, quad, 0)
        for k in range(n_chunks - n_chunks % nd, n_chunks):
            b = k % nd
            drain(b)
            if k + nd - 1 < n_chunks:
                issue(k + nd - 1, (k + nd - 1) % nd)
            process(k, b)

    return gk


def _make_scatter(epb, csz):
    epw = epb // 16
    n_full = epw // csz
    rem = epw % csz
    assert rem % 8 == 0
    trem = max(rem, 8)
    nd = 4
    mesh = plsc.VectorSubcoreMesh(core_axis_name="c", subcore_axis_name="s")

    @functools.partial(
        pl.kernel, mesh=mesh,
        out_type=jax.ShapeDtypeStruct((2 * _NM, _HID), F32),
        scratch_types=[
            pltpu.VMEM((nd, csz), jnp.int32),
            pltpu.VMEM((nd, csz, _HID), F32),
            pltpu.VMEM((trem,), jnp.int32),
            pltpu.VMEM((trem, _HID), F32),
            pltpu.VMEM_SHARED((_SC_R, _HID), F32),
            pltpu.SemaphoreType.DMA,
            pltpu.SemaphoreType.DMA,
            pltpu.SemaphoreType.DMA,
            pltpu.SemaphoreType.DMA,
        ],
    )
    def sk(m_hbm, dst_hbm, agg_hbm, idx_v, chunk_v, idx_t, chunk_t,
           shared, sem0, sem1, sem2, sem3):
        b = lax.axis_index("c")
        s = lax.axis_index("s")
        sems = (sem0, sem1, sem2, sem3)
        wbase = b * epb + s * epw

        zb = chunk_v.at[0]

        def zrow(r, carry):
            for j in range(_HID // 16):
                zb[r, pl.ds(j * 16, 16)] = jnp.zeros((16,), F32)
            return carry

        lax.fori_loop(0, csz, zrow, 0)
        stripe = _SC_R // 16
        for z in range(stripe // csz):
            pltpu.sync_copy(
                zb, shared.at[pl.ds(s * stripe + z * csz, csz)])
        plsc.subcore_barrier()

        def issue(k, bb):
            base = wbase + k * csz
            pltpu.async_copy(dst_hbm.at[pl.ds(base, csz)],
                             idx_v.at[bb], sems[bb])
            pltpu.async_copy(m_hbm.at[pl.ds(base, csz)],
                             chunk_v.at[bb], sems[bb])

        def drain(bb):
            pltpu.make_async_copy(dst_hbm.at[pl.ds(0, csz)],
                                  idx_v.at[bb], sems[bb]).wait()
            pltpu.make_async_copy(m_hbm.at[pl.ds(0, csz)],
                                  chunk_v.at[bb], sems[bb]).wait()

        def scat(bb):
            pltpu.sync_copy(chunk_v.at[bb], shared.at[idx_v.at[bb]],
                            add=True)

        for i in range(min(nd - 1, n_full)):
            issue(i, i % nd)

        def quad(q, carry):
            for bb in range(nd):
                k = q * nd + bb
                drain(bb)

                @pl.when(k + nd - 1 < n_full)
                def _nxt():
                    issue(k + nd - 1, (bb + nd - 1) % nd)

                scat(bb)
            return carry

        lax.fori_loop(0, n_full // nd, quad, 0)
        for k in range(n_full - n_full % nd, n_full):
            bb = k % nd
            drain(bb)
            if k + nd - 1 < n_full:
                issue(k + nd - 1, (k + nd - 1) % nd)
            scat(bb)
        if rem:
            tbase = wbase + n_full * csz
            pltpu.sync_copy(dst_hbm.at[pl.ds(tbase, rem)], idx_t)
            pltpu.sync_copy(m_hbm.at[pl.ds(tbase, rem)], chunk_t)
            pltpu.sync_copy(chunk_t, shared.at[idx_t], add=True)
        plsc.subcore_barrier()
        pltpu.sync_copy(shared.at[pl.ds(s * 624, 624)],
                        agg_hbm.at[pl.ds(b * _NM + s * 624, 624)])

        @pl.when(s == 0)
        def _tail():
            pltpu.sync_copy(shared.at[pl.ds(9984, 16)],
                            agg_hbm.at[pl.ds(b * _NM + 9984, 16)])

    return sk


def _b2d(v):
    return v.reshape(1, -1)


def _edge_block(s_tab, d_tab, src, dst_g, dst_s, w2, b2, g_pair, scatter,
                epb):
    half = 2 * epb
    aggs = []
    for i, g in enumerate(g_pair):
        sl = slice(i * half, (i + 1) * half)
        h = g(s_tab, d_tab, src[sl], dst_g[sl])
        m = _tc_call(_msg_body, [h], [w2, _b2d(b2)], 1, _BM_EDGE)
        aggs.append(scatter(m, dst_s[sl]))
    return jnp.concatenate(aggs, axis=0)


def kernel(madis_x, madis_lon, madis_lat, ex_lon, ex_lat, ex_x, params,
           edge_index, edge_index_e2m):
    p = params
    _gather_int = (_make_gather(_EI_PB, 80, 0), _make_gather(_EI_PB, 80, 2))
    _gather_ext = (_make_gather(_EE_PAD, 80, 0),
                   _make_gather(_EE_PAD, 80, 2))
    _scatter_int = _make_scatter(_EI_PB, 64)
    _scatter_ext = _make_scatter(_EE_PAD, 64)
    B, Nm, Tm, Fm = madis_x.shape
    u = madis_x.reshape(B * Nm, Tm * Fm)
    pos = jnp.concatenate([madis_lon, madis_lat], axis=2).reshape(B * Nm, 2)
    exf = ex_x.reshape(B * ex_x.shape[1], -1)
    ex_pos = jnp.concatenate([ex_lon, ex_lat], axis=2).reshape(-1, 2)

    src_i = edge_index[:, 0, :].reshape(-1)
    dst_i = edge_index[:, 1, :].reshape(-1)
    padw = ((0, 0), (0, _EE_PAD - _EE_PB))
    src_e = jnp.pad(edge_index_e2m[:, 0, :], padw).reshape(-1)
    dst_e_g = jnp.pad(edge_index_e2m[:, 1, :], padw).reshape(-1)
    dst_e_s = jnp.pad(edge_index_e2m[:, 1, :], padw,
                      constant_values=_DUMP).reshape(-1)

    in_x = _tc_call(
        _embed_body, [u, pos],
        [p["emb_W1"][:128], p["emb_W1"][128:130], _b2d(p["emb_b1"]),
         p["emb_W2"], _b2d(p["emb_b2"])], 1, _BM_NODE)

    for nm in ["ex1"] + ["in%d" % i for i in range(4)] + ["ex2"]:
        ew1 = p[nm + "_eW1"]
        if nm.startswith("ex"):
            s_tab, d_tab = _tc_call(
                _prep_ext_body, [exf, ex_pos, in_x, pos],
                [ew1[:64], ew1[64:192], ew1[192:194], _b2d(p[nm + "_eb1"])],
                2, _BM_NODE)
            agg = _edge_block(s_tab, d_tab, src_e, dst_e_g, dst_e_s,
                              p[nm + "_eW2"], p[nm + "_eb2"],
                              _gather_ext, _scatter_ext, _EE_PAD)
            in_x = _tc_call(
                _upd_ext_body, [in_x, agg],
                [p[nm + "_nW1"][:128], p[nm + "_nW1"][128:256],
                 _b2d(p[nm + "_nb1"]), p[nm + "_nW2"], _b2d(p[nm + "_nb2"])],
                1, _BM_NODE)
        else:
            s_tab, d_tab = _tc_call(
                _prep_int_body, [in_x, pos],
                [ew1[:128], ew1[128:256], ew1[256:258],
                 _b2d(p[nm + "_eb1"])], 2, _BM_NODE)
            agg = _edge_block(s_tab, d_tab, src_i, dst_i, dst_i,
                              p[nm + "_eW2"], p[nm + "_eb2"],
                              _gather_int, _scatter_int, _EI_PB)
            in_x = _tc_call(
                _upd_int_body, [in_x, agg, u],
                [p[nm + "_nW1"][:128], p[nm + "_nW1"][128:256],
                 p[nm + "_nW1"][256:384], _b2d(p[nm + "_nb1"]),
                 p[nm + "_nW2"], _b2d(p[nm + "_nb2"])], 1, _BM_NODE)

    w2p = jnp.zeros((_HID, _HID), F32).at[:, :2].set(p["out_W2"])
    b2p = jnp.zeros((_HID,), F32).at[:2].set(p["out_b2"])
    out = _tc_call(_out_body, [in_x],
                   [p["out_W1"], _b2d(p["out_b1"]), w2p, _b2d(b2p)],
                   1, _BM_NODE)
    return out[:, :2].reshape(B, Nm, 2)


# speedup vs baseline: 1.5735x; 1.0136x over previous
import functools

import jax
import jax.numpy as jnp
from jax import lax
from jax.experimental import pallas as pl
from jax.experimental.pallas import tpu as pltpu
from jax.experimental.pallas import tpu_sc as plsc

F32 = jnp.float32
_HID = 128
_B = 4
_NM = 10000
_N = _B * _NM
_EI_PB = 160000
_EI_PAD = 163840
_EE_PB = 40000
_EE_PAD = 40960
_C = 128
_SC_R = 10240
_DUMP = 10000
_BM_NODE = 2000
_BM_EDGE = 2560

_PREC = None


def _dot(a, b):
    return jnp.dot(a, b, preferred_element_type=F32, precision=_PREC)


def _posmul(pos2, wp):
    return pos2[:, 0:1] * wp[0:1, :] + pos2[:, 1:2] * wp[1:2, :]


def _tc_call(body, row_args, const_args, n_out, bm):
    n = row_args[0].shape[0]
    grid = (n // bm,)
    in_specs = (
        [pl.BlockSpec((bm, a.shape[1]), lambda i: (i, 0)) for a in row_args]
        + [pl.BlockSpec(a.shape, lambda i, nd=a.ndim: (0,) * nd)
           for a in const_args]
    )
    out_shape = [jax.ShapeDtypeStruct((n, _HID), F32) for _ in range(n_out)]
    out_specs = [pl.BlockSpec((bm, _HID), lambda i: (i, 0))
                 for _ in range(n_out)]
    res = pl.pallas_call(
        body,
        grid=grid,
        in_specs=in_specs,
        out_shape=out_shape,
        out_specs=out_specs,
    )(*row_args, *const_args)
    return res if n_out > 1 else res[0]


def _embed_body(u, pos, w1u, w1p, b1, w2, b2, o):
    h = _dot(u[...], w1u[...]) + _posmul(pos[...], w1p[...]) + b1[...]
    h = jnp.tanh(h)
    o[...] = jnp.tanh(_dot(h, w2[...]) + b2[...])


def _prep_int_body(x, pos, ws, wd, wp, b1, s_o, d_o):
    pw = _posmul(pos[...], wp[...])
    s_o[...] = _dot(x[...], ws[...]) - pw
    d_o[...] = _dot(x[...], wd[...]) + pw + b1[...]


def _prep_ext_body(exf, ex_pos, x, pos, wsx, wd, wp, b1, s_o, d_o):
    s_o[...] = _dot(exf[...], wsx[...]) - _posmul(ex_pos[...], wp[...])
    d_o[...] = _dot(x[...], wd[...]) + _posmul(pos[...], wp[...]) + b1[...]


def _msg_body(h, w2, b2, o):
    o[...] = jnp.tanh(_dot(jnp.tanh(h[...]), w2[...]) + b2[...])


def _upd_int_body(x, agg, u, wa, wb, wc, b1, w2, b2, o):
    h = (_dot(x[...], wa[...]) + _dot(agg[...], wb[...])
         + _dot(u[...], wc[...]) + b1[...])
    o[...] = x[...] + _dot(jnp.tanh(h), w2[...]) + b2[...]


def _upd_ext_body(x, agg, wa, wb, b1, w2, b2, o):
    h = _dot(x[...], wa[...]) + _dot(agg[...], wb[...]) + b1[...]
    o[...] = x[...] + _dot(jnp.tanh(h), w2[...]) + b2[...]


def _out_body(x, w1, b1, w2, b2, o):
    h = jnp.tanh(_dot(x[...], w1[...]) + b1[...])
    o[...] = _dot(h, w2[...]) + b2[...]


def _make_gather(epb, gcsz, b0):
    epw = epb // 16
    n_chunks = epw // gcsz
    nd = 4
    mesh = plsc.VectorSubcoreMesh(core_axis_name="c", subcore_axis_name="s")

    @functools.partial(
        pl.kernel, mesh=mesh,
        out_type=jax.ShapeDtypeStruct((2 * epb, _HID), F32),
        scratch_types=[
            pltpu.VMEM((epw,), jnp.int32),
            pltpu.VMEM((epw,), jnp.int32),
            pltpu.VMEM((nd, gcsz, _HID), F32),
            pltpu.VMEM((nd, gcsz, _HID), F32),
            pltpu.SemaphoreType.DMA,
            pltpu.SemaphoreType.DMA,
            pltpu.SemaphoreType.DMA,
            pltpu.SemaphoreType.DMA,
        ],
    )
    def gk(s_hbm, d_hbm, src_hbm, dst_hbm, h_hbm,
           idx_s, idx_d, buf_s, buf_d, sem0, sem1, sem2, sem3):
        wid = lax.axis_index("s") * 2 + lax.axis_index("c")
        batch = wid // 16
        lane = wid % 16
        shift = (batch + b0) * _NM
        wbase = batch * epb + lane * epw

        pltpu.sync_copy(src_hbm.at[pl.ds(wbase, epw)], idx_s)
        pltpu.sync_copy(dst_hbm.at[pl.ds(wbase, epw)], idx_d)

        def sh(i, carry):
            sl = pl.ds(i * 16, 16)
            idx_s[sl] = idx_s[sl] + shift
            idx_d[sl] = idx_d[sl] + shift
            return carry

        lax.fori_loop(0, epw // 16, sh, 0)

        sems = (sem0, sem1, sem2, sem3)

        def issue(k, b):
            pltpu.async_copy(s_hbm.at[idx_s.at[pl.ds(k * gcsz, gcsz)]],
                             buf_s.at[b], sems[b])
            pltpu.async_copy(d_hbm.at[idx_d.at[pl.ds(k * gcsz, gcsz)]],
                             buf_d.at[b], sems[b])

        def drain(b):
            pltpu.make_async_copy(h_hbm.at[pl.ds(0, gcsz)],
                                  buf_s.at[b], sems[b]).wait()
            pltpu.make_async_copy(h_hbm.at[pl.ds(0, gcsz)],
                                  buf_d.at[b], sems[b]).wait()

        def process(k, b):
            bs = buf_s.at[b]
            bd = buf_d.at[b]

            def addrow(r, c2):
                for j in range(_HID // 16):
                    sl = pl.ds(j * 16, 16)
                    bs[r, sl] = bs[r, sl] + bd[r, sl]
                return c2

            lax.fori_loop(0, gcsz, addrow, 0)
            pltpu.sync_copy(bs, h_hbm.at[pl.ds(wbase + k * gcsz, gcsz)])

        for i in range(min(nd - 1, n_chunks)):
            issue(i, i % nd)

        def quad(q, carry):
            for b in range(nd):
                k = q * nd + b
                drain(b)

                @pl.when(k + nd - 1 < n_chunks)
                def _nxt():
                    issue(k + nd - 1, (b + nd - 1) % nd)

                process(k, b)
            return carry

        lax.fori_loop(0, n_chunks // nd, quad, 0)
        for k in range(n_chunks - n_chunks % nd, n_chunks):
            b = k % nd
            drain(b)
            if k + nd - 1 < n_chunks:
                issue(k + nd - 1, (k + nd - 1) % nd)
            process(k, b)

    return gk


def _make_scatter(epb, csz):
    epw = epb // 16
    n_full = epw // csz
    rem = epw % csz
    assert rem % 8 == 0
    trem = max(rem, 8)
    nd = 4
    mesh = plsc.VectorSubcoreMesh(core_axis_name="c", subcore_axis_name="s")

    @functools.partial(
        pl.kernel, mesh=mesh,
        out_type=jax.ShapeDtypeStruct((2 * _NM, _HID), F32),
        scratch_types=[
            pltpu.VMEM((nd, csz), jnp.int32),
            pltpu.VMEM((nd, csz, _HID), F32),
            pltpu.VMEM((trem,), jnp.int32),
            pltpu.VMEM((trem, _HID), F32),
            pltpu.VMEM_SHARED((_SC_R, _HID), F32),
            pltpu.SemaphoreType.DMA,
            pltpu.SemaphoreType.DMA,
            pltpu.SemaphoreType.DMA,
            pltpu.SemaphoreType.DMA,
        ],
    )
    def sk(m_hbm, dst_hbm, agg_hbm, idx_v, chunk_v, idx_t, chunk_t,
           shared, sem0, sem1, sem2, sem3):
        b = lax.axis_index("c")
        s = lax.axis_index("s")
        sems = (sem0, sem1, sem2, sem3)
        wbase = b * epb + s * epw

        zb = chunk_v.at[0]

        def zrow(r, carry):
            for j in range(_HID // 16):
                zb[r, pl.ds(j * 16, 16)] = jnp.zeros((16,), F32)
            return carry

        lax.fori_loop(0, csz, zrow, 0)
        stripe = _SC_R // 16
        for z in range(stripe // csz):
            pltpu.sync_copy(
                zb, shared.at[pl.ds(s * stripe + z * csz, csz)])
        plsc.subcore_barrier()

        def issue(k, bb):
            base = wbase + k * csz
            pltpu.async_copy(dst_hbm.at[pl.ds(base, csz)],
                             idx_v.at[bb], sems[bb])
            pltpu.async_copy(m_hbm.at[pl.ds(base, csz)],
                             chunk_v.at[bb], sems[bb])

        def drain(bb):
            pltpu.make_async_copy(dst_hbm.at[pl.ds(0, csz)],
                                  idx_v.at[bb], sems[bb]).wait()
            pltpu.make_async_copy(m_hbm.at[pl.ds(0, csz)],
                                  chunk_v.at[bb], sems[bb]).wait()

        def scat(bb):
            pltpu.sync_copy(chunk_v.at[bb], shared.at[idx_v.at[bb]],
                            add=True)

        for i in range(min(nd - 1, n_full)):
            issue(i, i % nd)

        def quad(q, carry):
            for bb in range(nd):
                k = q * nd + bb
                drain(bb)

                @pl.when(k + nd - 1 < n_full)
                def _nxt():
                    issue(k + nd - 1, (bb + nd - 1) % nd)

                scat(bb)
            return carry

        lax.fori_loop(0, n_full // nd, quad, 0)
        for k in range(n_full - n_full % nd, n_full):
            bb = k % nd
            drain(bb)
            if k + nd - 1 < n_full:
                issue(k + nd - 1, (k + nd - 1) % nd)
            scat(bb)
        if rem:
            tbase = wbase + n_full * csz
            pltpu.sync_copy(dst_hbm.at[pl.ds(tbase, rem)], idx_t)
            pltpu.sync_copy(m_hbm.at[pl.ds(tbase, rem)], chunk_t)
            pltpu.sync_copy(chunk_t, shared.at[idx_t], add=True)
        plsc.subcore_barrier()
        pltpu.sync_copy(shared.at[pl.ds(s * 624, 624)],
                        agg_hbm.at[pl.ds(b * _NM + s * 624, 624)])

        @pl.when(s == 0)
        def _tail():
            pltpu.sync_copy(shared.at[pl.ds(9984, 16)],
                            agg_hbm.at[pl.ds(b * _NM + 9984, 16)])

    return sk


def _b2d(v):
    return v.reshape(1, -1)


def _edge_block(s_tab, d_tab, src, dst_g, dst_s, w2, b2, g_pair, scatter,
                epb):
    half = 2 * epb
    bm = 4000 if half % 4000 == 0 else _BM_EDGE
    aggs = []
    for i, g in enumerate(g_pair):
        sl = slice(i * half, (i + 1) * half)
        h = g(s_tab, d_tab, src[sl], dst_g[sl])
        m = _tc_call(_msg_body, [h], [w2, _b2d(b2)], 1, bm)
        aggs.append(scatter(m, dst_s[sl]))
    return jnp.concatenate(aggs, axis=0)


def kernel(madis_x, madis_lon, madis_lat, ex_lon, ex_lat, ex_x, params,
           edge_index, edge_index_e2m):
    p = params
    _gather_int = (_make_gather(_EI_PB, 80, 0), _make_gather(_EI_PB, 80, 2))
    _gather_ext = (_make_gather(_EE_PAD, 80, 0),
                   _make_gather(_EE_PAD, 80, 2))
    _scatter_int = _make_scatter(_EI_PB, 64)
    _scatter_ext = _make_scatter(_EE_PAD, 64)
    B, Nm, Tm, Fm = madis_x.shape
    u = madis_x.reshape(B * Nm, Tm * Fm)
    pos = jnp.concatenate([madis_lon, madis_lat], axis=2).reshape(B * Nm, 2)
    exf = ex_x.reshape(B * ex_x.shape[1], -1)
    ex_pos = jnp.concatenate([ex_lon, ex_lat], axis=2).reshape(-1, 2)

    src_i = edge_index[:, 0, :].reshape(-1)
    dst_i = edge_index[:, 1, :].reshape(-1)
    padw = ((0, 0), (0, _EE_PAD - _EE_PB))
    src_e = jnp.pad(edge_index_e2m[:, 0, :], padw).reshape(-1)
    dst_e_g = jnp.pad(edge_index_e2m[:, 1, :], padw).reshape(-1)
    dst_e_s = jnp.pad(edge_index_e2m[:, 1, :], padw,
                      constant_values=_DUMP).reshape(-1)

    in_x = _tc_call(
        _embed_body, [u, pos],
        [p["emb_W1"][:128], p["emb_W1"][128:130], _b2d(p["emb_b1"]),
         p["emb_W2"], _b2d(p["emb_b2"])], 1, _BM_NODE)

    for nm in ["ex1"] + ["in%d" % i for i in range(4)] + ["ex2"]:
        ew1 = p[nm + "_eW1"]
        if nm.startswith("ex"):
            s_tab, d_tab = _tc_call(
                _prep_ext_body, [exf, ex_pos, in_x, pos],
                [ew1[:64], ew1[64:192], ew1[192:194], _b2d(p[nm + "_eb1"])],
                2, _BM_NODE)
            agg = _edge_block(s_tab, d_tab, src_e, dst_e_g, dst_e_s,
                              p[nm + "_eW2"], p[nm + "_eb2"],
                              _gather_ext, _scatter_ext, _EE_PAD)
            in_x = _tc_call(
                _upd_ext_body, [in_x, agg],
                [p[nm + "_nW1"][:128], p[nm + "_nW1"][128:256],
                 _b2d(p[nm + "_nb1"]), p[nm + "_nW2"], _b2d(p[nm + "_nb2"])],
                1, _BM_NODE)
        else:
            s_tab, d_tab = _tc_call(
                _prep_int_body, [in_x, pos],
                [ew1[:128], ew1[128:256], ew1[256:258],
                 _b2d(p[nm + "_eb1"])], 2, _BM_NODE)
            agg = _edge_block(s_tab, d_tab, src_i, dst_i, dst_i,
                              p[nm + "_eW2"], p[nm + "_eb2"],
                              _gather_int, _scatter_int, _EI_PB)
            in_x = _tc_call(
                _upd_int_body, [in_x, agg, u],
                [p[nm + "_nW1"][:128], p[nm + "_nW1"][128:256],
                 p[nm + "_nW1"][256:384], _b2d(p[nm + "_nb1"]),
                 p[nm + "_nW2"], _b2d(p[nm + "_nb2"])], 1, _BM_NODE)

    w2p = jnp.zeros((_HID, _HID), F32).at[:, :2].set(p["out_W2"])
    b2p = jnp.zeros((_HID,), F32).at[:2].set(p["out_b2"])
    out = _tc_call(_out_body, [in_x],
                   [p["out_W1"], _b2d(p["out_b1"]), w2p, _b2d(b2p)],
                   1, _BM_NODE)
    return out[:, :2].reshape(B, Nm, 2)
